# Initial kernel scaffold; baseline (speedup 1.0000x reference)
import functools

import jax
import jax.numpy as jnp
from jax import lax
from jax.experimental import pallas as pl
from jax.experimental.pallas import tpu as pltpu
from jax.experimental.pallas import tpu_sc as plsc

B = 16384
D = 64
NC = 2
NS = 16
NW = NC * NS
L = 16
B_PER_W = B // NW
CHUNK = 128
N_CHUNKS = B_PER_W // CHUNK


def _sc_gather_mul_build():
    mesh = plsc.VectorSubcoreMesh(core_axis_name="c", subcore_axis_name="s")

    @functools.partial(
        pl.kernel,
        mesh=mesh,
        out_type=jax.ShapeDtypeStruct((B, D), jnp.float32),
        scratch_types=[
            pltpu.VMEM((N_CHUNKS, CHUNK), jnp.int32),
            pltpu.VMEM((N_CHUNKS, CHUNK), jnp.int32),
            pltpu.VMEM((B_PER_W, D), jnp.float32),
            pltpu.VMEM((B_PER_W, D), jnp.float32),
            pltpu.SemaphoreType.DMA,
        ],
    )
    def sc_kernel(uidx_hbm, iidx_hbm, utab_hbm, itab_hbm, out_hbm,
                  uidx_v, iidx_v, urows, irows, sem):
        wid = lax.axis_index("s") * NC + lax.axis_index("c")
        base = wid * B_PER_W
        pltpu.sync_copy(uidx_hbm.at[wid], uidx_v)
        pltpu.sync_copy(iidx_hbm.at[wid], iidx_v)
        copies = []
        for j in range(N_CHUNKS):
            dst = pl.ds(j * CHUNK, CHUNK)
            copies.append(pltpu.make_async_copy(
                utab_hbm.at[uidx_v.at[j]], urows.at[dst], sem))
            copies.append(pltpu.make_async_copy(
                itab_hbm.at[iidx_v.at[j]], irows.at[dst], sem))
        for c in copies:
            c.start()
        for c in copies:
            c.wait()

        def body(i, carry):
            for r in range(4):
                row = i * 4 + r
                for cidx in range(D // L):
                    sl = pl.ds(cidx * L, L)
                    urows[row, sl] = urows[row, sl] * irows[row, sl]
            return carry

        lax.fori_loop(0, B_PER_W // 4, body, 0)
        pltpu.sync_copy(urows, out_hbm.at[pl.ds(base, B_PER_W)])

    return sc_kernel


_sc_gather_mul = _sc_gather_mul_build()

ROWS_BLK = 2048


def _mlp_body(h_ref, w1_ref, b1_ref, w2_ref, b2_ref, o_ref):
    h = h_ref[...]
    z = jnp.dot(h, w1_ref[...], preferred_element_type=jnp.float32)
    z = jnp.maximum(z + b1_ref[...], 0.0)
    o = jnp.dot(z, w2_ref[...], preferred_element_type=jnp.float32)
    o = o + b2_ref[...]
    o_ref[...] = 1.0 / (1.0 + jnp.exp(-o))


@jax.jit
def _tc_mlp(h, W1, b1, W2, b2):
    grid = (B // ROWS_BLK,)
    return pl.pallas_call(
        _mlp_body,
        grid=grid,
        in_specs=[
            pl.BlockSpec((ROWS_BLK, D), lambda i: (i, 0)),
            pl.BlockSpec((D, 16), lambda i: (0, 0)),
            pl.BlockSpec((1, 16), lambda i: (0, 0)),
            pl.BlockSpec((16, 1), lambda i: (0, 0)),
            pl.BlockSpec((1, 1), lambda i: (0, 0)),
        ],
        out_specs=pl.BlockSpec((ROWS_BLK, 1), lambda i: (i, 0)),
        out_shape=jax.ShapeDtypeStruct((B, 1), jnp.float32),
    )(h, W1, b1, W2, b2)


@jax.jit
def kernel(group_inputs, user_inputs, item_inputs, user_table, item_table,
           W1, b1, W2, b2):
    del group_inputs
    uidx = user_inputs.astype(jnp.int32).reshape(NW, N_CHUNKS, CHUNK)
    iidx = item_inputs.astype(jnp.int32).reshape(NW, N_CHUNKS, CHUNK)
    h = _sc_gather_mul(uidx, iidx, user_table, item_table)
    return _tc_mlp(h, W1, b1.reshape(1, 16), W2, b2.reshape(1, 1))


# speedup vs baseline: 1.1768x; 1.1768x over previous
import functools

import jax
import jax.numpy as jnp
from jax import lax
from jax.experimental import pallas as pl
from jax.experimental.pallas import tpu as pltpu
from jax.experimental.pallas import tpu_sc as plsc

B = 16384
D = 64
NC = 2
NS = 16
NW = NC * NS
L = 16
B_PER_W = B // NW
NBUF = 4


def _sc_gather_build():
    mesh = plsc.VectorSubcoreMesh(core_axis_name="c", subcore_axis_name="s")

    @functools.partial(
        pl.kernel,
        mesh=mesh,
        compiler_params=pltpu.CompilerParams(use_tc_tiling_on_sc=True, needs_layout_passes=False),
        out_type=(jax.ShapeDtypeStruct((B // 2, 2 * D), jnp.float32),
                  jax.ShapeDtypeStruct((B // 2, 2 * D), jnp.float32)),
        scratch_types=[
            pltpu.VMEM((B_PER_W,), jnp.int32),
            pltpu.VMEM((NBUF, D, 128), jnp.float32),
            pltpu.VMEM((B_PER_W // 2, 2 * D), jnp.float32),
            pltpu.SemaphoreType.DMA,
        ],
    )
    def sc_kernel(uidx_hbm, iidx_hbm, utabT_hbm, itabT_hbm, hu_hbm, hi_hbm,
                  idx_sm, wins, rows, sem):
        wid = lax.axis_index("s") * NC + lax.axis_index("c")
        base = pl.multiple_of(wid * B_PER_W, B_PER_W)
        lanes = lax.iota(jnp.int32, 16)

        def phase(idx_hbm, tab_hbm, out_hbm):
            pltpu.sync_copy(idx_hbm.at[pl.ds(base, B_PER_W)], idx_sm)

            def win_copy(val, buf):
                t = lax.shift_right_logical(val, 7)
                off = pl.multiple_of(t * 128, 128)
                return pltpu.make_async_copy(
                    tab_hbm.at[:, pl.ds(off, 128)], wins.at[buf], sem)

            vec0 = idx_sm[pl.ds(0, L)]
            for j in range(NBUF):
                win_copy(vec0[j], j).start()

            def group(gg, carry):
                voff = pl.multiple_of(gg * L, L)
                vec = idx_sm[pl.ds(voff, L)]
                noff = pl.multiple_of(
                    jnp.minimum(gg * L + L, B_PER_W - L), L)
                vecn = idx_sm[pl.ds(noff, L)]
                for j in range(L):
                    i = gg * L + j
                    val = vec[j]
                    win_copy(val, j % NBUF).wait()
                    m = lax.bitwise_and(val, 127)
                    mv = jnp.full((16,), 0, jnp.int32) + m
                    row = lax.shift_right_logical(i, 1)
                    colb = lax.bitwise_and(i, 1) * D
                    for c in range(D // L):
                        v = plsc.load_gather(
                            wins.at[j % NBUF], [c * L + lanes, mv])
                        rows[row, pl.ds(colb + c * L, L)] = v
                    nval = vec[j + NBUF] if j < L - NBUF else vecn[j - (L - NBUF)]
                    nxt = i + NBUF

                    @pl.when(nxt < B_PER_W)
                    def _():
                        win_copy(nval, j % NBUF).start()
                return carry

            lax.fori_loop(0, B_PER_W // L, group, 0)
            obase = pl.multiple_of(wid * (B_PER_W // 2), B_PER_W // 2)
            pltpu.sync_copy(rows, out_hbm.at[pl.ds(obase, B_PER_W // 2)])

        phase(uidx_hbm, utabT_hbm, hu_hbm)
        phase(iidx_hbm, itabT_hbm, hi_hbm)

    return sc_kernel


_sc_gather = _sc_gather_build()

ROW_BLK = 2048


def _mlp_body(hu_ref, hi_ref, w1d_ref, b1d_ref, w2d_ref, b2d_ref, o_ref):
    h = hu_ref[...] * hi_ref[...]
    z = jnp.dot(h, w1d_ref[...], preferred_element_type=jnp.float32)
    z = jnp.maximum(z + b1d_ref[...], 0.0)
    o = jnp.dot(z, w2d_ref[...], preferred_element_type=jnp.float32)
    o = o + b2d_ref[...]
    o_ref[...] = 1.0 / (1.0 + jnp.exp(-o))


@jax.jit
def _tc_mlp(hu, hi, W1d, b1d, W2d, b2d):
    grid = ((B // 2) // ROW_BLK,)
    return pl.pallas_call(
        _mlp_body,
        grid=grid,
        in_specs=[
            pl.BlockSpec((ROW_BLK, 2 * D), lambda i: (i, 0)),
            pl.BlockSpec((ROW_BLK, 2 * D), lambda i: (i, 0)),
            pl.BlockSpec((2 * D, 32), lambda i: (0, 0)),
            pl.BlockSpec((1, 32), lambda i: (0, 0)),
            pl.BlockSpec((32, 2), lambda i: (0, 0)),
            pl.BlockSpec((1, 2), lambda i: (0, 0)),
        ],
        out_specs=pl.BlockSpec((ROW_BLK, 2), lambda i: (i, 0)),
        out_shape=jax.ShapeDtypeStruct((B // 2, 2), jnp.float32),
    )(hu, hi, W1d, b1d, W2d, b2d)


@jax.jit
def kernel(group_inputs, user_inputs, item_inputs, user_table, item_table,
           W1, b1, W2, b2):
    del group_inputs
    uidx = user_inputs.astype(jnp.int32)
    iidx = item_inputs.astype(jnp.int32)
    hu, hi = _sc_gather(uidx, iidx, user_table.T, item_table.T)
    zero = jnp.zeros((D, 16), jnp.float32)
    W1d = jnp.concatenate(
        [jnp.concatenate([W1, zero], axis=1),
         jnp.concatenate([zero, W1], axis=1)], axis=0)
    b1d = jnp.concatenate([b1, b1]).reshape(1, 32)
    zero2 = jnp.zeros((16, 1), jnp.float32)
    W2d = jnp.concatenate(
        [jnp.concatenate([W2, zero2], axis=1),
         jnp.concatenate([zero2, W2], axis=1)], axis=0)
    b2d = jnp.concatenate([b2, b2]).reshape(1, 2)
    o = _tc_mlp(hu, hi, W1d, b1d, W2d, b2d)
    return o.reshape(B, 1)


# speedup vs baseline: 1.2197x; 1.0365x over previous
import functools

import jax
import jax.numpy as jnp
from jax import lax
from jax.experimental import pallas as pl
from jax.experimental.pallas import tpu as pltpu
from jax.experimental.pallas import tpu_sc as plsc

B = 16384
D = 64
NC = 2
NS = 16
NW = NC * NS
L = 16
B_PER_W = B // NW
NBUF = 8


def _sc_gather_build():
    mesh = plsc.VectorSubcoreMesh(core_axis_name="c", subcore_axis_name="s")

    @functools.partial(
        pl.kernel,
        mesh=mesh,
        compiler_params=pltpu.CompilerParams(use_tc_tiling_on_sc=True, needs_layout_passes=False),
        out_type=(jax.ShapeDtypeStruct((B // 2, 2 * D), jnp.float32),
                  jax.ShapeDtypeStruct((B // 2, 2 * D), jnp.float32)),
        scratch_types=[
            pltpu.VMEM((B_PER_W,), jnp.int32),
            pltpu.VMEM((NBUF, D, 128), jnp.float32),
            pltpu.VMEM((B_PER_W // 2, 2 * D), jnp.float32),
            pltpu.SemaphoreType.DMA,
        ],
    )
    def sc_kernel(uidx_hbm, iidx_hbm, utabT_hbm, itabT_hbm, hu_hbm, hi_hbm,
                  idx_sm, wins, rows, sem):
        wid = lax.axis_index("s") * NC + lax.axis_index("c")
        base = pl.multiple_of(wid * B_PER_W, B_PER_W)
        lanes = lax.iota(jnp.int32, 16)

        def phase(idx_hbm, tab_hbm, out_hbm):
            pltpu.sync_copy(idx_hbm.at[pl.ds(base, B_PER_W)], idx_sm)

            def win_copy(val, buf):
                t = lax.shift_right_logical(val, 7)
                off = pl.multiple_of(t * 128, 128)
                return pltpu.make_async_copy(
                    tab_hbm.at[:, pl.ds(off, 128)], wins.at[buf], sem)

            vec0 = idx_sm[pl.ds(0, L)]
            for j in range(NBUF):
                win_copy(vec0[j], j).start()

            def group(gg, carry):
                voff = pl.multiple_of(gg * L, L)
                vec = idx_sm[pl.ds(voff, L)]
                noff = pl.multiple_of(
                    jnp.minimum(gg * L + L, B_PER_W - L), L)
                vecn = idx_sm[pl.ds(noff, L)]
                for j in range(L):
                    i = gg * L + j
                    val = vec[j]
                    win_copy(val, j % NBUF).wait()
                    m = lax.bitwise_and(val, 127)
                    mv = jnp.full((16,), 0, jnp.int32) + m
                    row = lax.shift_right_logical(i, 1)
                    colb = lax.bitwise_and(i, 1) * D
                    for c in range(D // L):
                        v = plsc.load_gather(
                            wins.at[j % NBUF], [c * L + lanes, mv])
                        rows[row, pl.ds(colb + c * L, L)] = v
                    nval = vec[j + NBUF] if j < L - NBUF else vecn[j - (L - NBUF)]
                    nxt = i + NBUF

                    @pl.when(nxt < B_PER_W)
                    def _():
                        win_copy(nval, j % NBUF).start()
                return carry

            lax.fori_loop(0, B_PER_W // L, group, 0)
            obase = pl.multiple_of(wid * (B_PER_W // 2), B_PER_W // 2)
            pltpu.sync_copy(rows, out_hbm.at[pl.ds(obase, B_PER_W // 2)])

        phase(uidx_hbm, utabT_hbm, hu_hbm)
        phase(iidx_hbm, itabT_hbm, hi_hbm)

    return sc_kernel


_sc_gather = _sc_gather_build()

ROW_BLK = 2048


def _mlp_body(hu_ref, hi_ref, w1d_ref, b1d_ref, w2d_ref, b2d_ref, o_ref):
    h = hu_ref[...] * hi_ref[...]
    z = jnp.dot(h, w1d_ref[...], preferred_element_type=jnp.float32)
    z = jnp.maximum(z + b1d_ref[...], 0.0)
    o = jnp.dot(z, w2d_ref[...], preferred_element_type=jnp.float32)
    o = o + b2d_ref[...]
    o_ref[...] = 1.0 / (1.0 + jnp.exp(-o))


@jax.jit
def _tc_mlp(hu, hi, W1d, b1d, W2d, b2d):
    grid = ((B // 2) // ROW_BLK,)
    return pl.pallas_call(
        _mlp_body,
        grid=grid,
        in_specs=[
            pl.BlockSpec((ROW_BLK, 2 * D), lambda i: (i, 0)),
            pl.BlockSpec((ROW_BLK, 2 * D), lambda i: (i, 0)),
            pl.BlockSpec((2 * D, 32), lambda i: (0, 0)),
            pl.BlockSpec((1, 32), lambda i: (0, 0)),
            pl.BlockSpec((32, 2), lambda i: (0, 0)),
            pl.BlockSpec((1, 2), lambda i: (0, 0)),
        ],
        out_specs=pl.BlockSpec((ROW_BLK, 2), lambda i: (i, 0)),
        out_shape=jax.ShapeDtypeStruct((B // 2, 2), jnp.float32),
    )(hu, hi, W1d, b1d, W2d, b2d)


@jax.jit
def kernel(group_inputs, user_inputs, item_inputs, user_table, item_table,
           W1, b1, W2, b2):
    del group_inputs
    uidx = user_inputs.astype(jnp.int32)
    iidx = item_inputs.astype(jnp.int32)
    hu, hi = _sc_gather(uidx, iidx, user_table.T, item_table.T)
    zero = jnp.zeros((D, 16), jnp.float32)
    W1d = jnp.concatenate(
        [jnp.concatenate([W1, zero], axis=1),
         jnp.concatenate([zero, W1], axis=1)], axis=0)
    b1d = jnp.concatenate([b1, b1]).reshape(1, 32)
    zero2 = jnp.zeros((16, 1), jnp.float32)
    W2d = jnp.concatenate(
        [jnp.concatenate([W2, zero2], axis=1),
         jnp.concatenate([zero2, W2], axis=1)], axis=0)
    b2d = jnp.concatenate([b2, b2]).reshape(1, 2)
    o = _tc_mlp(hu, hi, W1d, b1d, W2d, b2d)
    return o.reshape(B, 1)


# speedup vs baseline: 1.2503x; 1.0251x over previous
import functools

import jax
import jax.numpy as jnp
from jax import lax
from jax.experimental import pallas as pl
from jax.experimental.pallas import tpu as pltpu
from jax.experimental.pallas import tpu_sc as plsc

B = 16384
D = 64
NC = 2
NS = 16
NW = NC * NS
L = 16
B_PER_W = B // NW
NBUF = 4


def _sc_gather_mul_build():
    mesh = plsc.VectorSubcoreMesh(core_axis_name="c", subcore_axis_name="s")

    @functools.partial(
        pl.kernel,
        mesh=mesh,
        compiler_params=pltpu.CompilerParams(
            use_tc_tiling_on_sc=True, needs_layout_passes=False),
        out_type=jax.ShapeDtypeStruct((B // 2, 2 * D), jnp.float32),
        scratch_types=[
            pltpu.VMEM((B_PER_W,), jnp.int32),
            pltpu.VMEM((B_PER_W,), jnp.int32),
            pltpu.VMEM((NBUF, D, 128), jnp.float32),
            pltpu.VMEM((NBUF, D, 128), jnp.float32),
            pltpu.VMEM((B_PER_W // 2, 2 * D), jnp.float32),
            pltpu.SemaphoreType.DMA,
        ],
    )
    def sc_kernel(uidx_hbm, iidx_hbm, utabT_hbm, itabT_hbm, h_hbm,
                  uidx_vm, iidx_vm, uwins, iwins, rows, sem):
        wid = lax.axis_index("s") * NC + lax.axis_index("c")
        base = pl.multiple_of(wid * B_PER_W, B_PER_W)
        lanes = lax.iota(jnp.int32, 16)
        pltpu.sync_copy(uidx_hbm.at[pl.ds(base, B_PER_W)], uidx_vm)
        pltpu.sync_copy(iidx_hbm.at[pl.ds(base, B_PER_W)], iidx_vm)

        def win_copy(tab_hbm, wins, val, buf):
            t = lax.shift_right_logical(val, 7)
            off = pl.multiple_of(t * 128, 128)
            return pltpu.make_async_copy(
                tab_hbm.at[:, pl.ds(off, 128)], wins.at[buf], sem)

        uvec0 = uidx_vm[pl.ds(0, L)]
        ivec0 = iidx_vm[pl.ds(0, L)]
        for j in range(NBUF):
            win_copy(utabT_hbm, uwins, uvec0[j], j).start()
            win_copy(itabT_hbm, iwins, ivec0[j], j).start()

        def group(gg, carry):
            voff = pl.multiple_of(gg * L, L)
            uvec = uidx_vm[pl.ds(voff, L)]
            ivec = iidx_vm[pl.ds(voff, L)]
            noff = pl.multiple_of(
                jnp.minimum(gg * L + L, B_PER_W - L), L)
            uvecn = uidx_vm[pl.ds(noff, L)]
            ivecn = iidx_vm[pl.ds(noff, L)]
            for j in range(L):
                i = gg * L + j
                s = j % NBUF
                uval = uvec[j]
                ival = ivec[j]
                win_copy(utabT_hbm, uwins, uval, s).wait()
                win_copy(itabT_hbm, iwins, ival, s).wait()
                umv = jnp.full((16,), 0, jnp.int32) + lax.bitwise_and(
                    uval, 127)
                imv = jnp.full((16,), 0, jnp.int32) + lax.bitwise_and(
                    ival, 127)
                row = lax.shift_right_logical(i, 1)
                colb = lax.bitwise_and(i, 1) * D
                for c in range(D // L):
                    kv = c * L + lanes
                    vu = plsc.load_gather(uwins.at[s], [kv, umv])
                    vi = plsc.load_gather(iwins.at[s], [kv, imv])
                    rows[row, pl.ds(colb + c * L, L)] = vu * vi
                if j < L - NBUF:
                    unval, inval = uvec[j + NBUF], ivec[j + NBUF]
                else:
                    unval = uvecn[j - (L - NBUF)]
                    inval = ivecn[j - (L - NBUF)]
                nxt = i + NBUF

                @pl.when(nxt < B_PER_W)
                def _():
                    win_copy(utabT_hbm, uwins, unval, s).start()
                    win_copy(itabT_hbm, iwins, inval, s).start()
            return carry

        lax.fori_loop(0, B_PER_W // L, group, 0)
        obase = pl.multiple_of(wid * (B_PER_W // 2), B_PER_W // 2)
        pltpu.sync_copy(rows, h_hbm.at[pl.ds(obase, B_PER_W // 2)])

    return sc_kernel


_sc_gather_mul = _sc_gather_mul_build()

ROW_BLK = 2048


def _mlp_body(h_ref, w1d_ref, b1d_ref, w2d_ref, b2d_ref, o_ref):
    h = h_ref[...]
    z = jnp.dot(h, w1d_ref[...], preferred_element_type=jnp.float32)
    z = jnp.maximum(z + b1d_ref[...], 0.0)
    o = jnp.dot(z, w2d_ref[...], preferred_element_type=jnp.float32)
    o = o + b2d_ref[...]
    o_ref[...] = 1.0 / (1.0 + jnp.exp(-o))


@jax.jit
def _tc_mlp(h, W1d, b1d, W2d, b2d):
    grid = ((B // 2) // ROW_BLK,)
    return pl.pallas_call(
        _mlp_body,
        grid=grid,
        in_specs=[
            pl.BlockSpec((ROW_BLK, 2 * D), lambda i: (i, 0)),
            pl.BlockSpec((2 * D, 32), lambda i: (0, 0)),
            pl.BlockSpec((1, 32), lambda i: (0, 0)),
            pl.BlockSpec((32, 2), lambda i: (0, 0)),
            pl.BlockSpec((1, 2), lambda i: (0, 0)),
        ],
        out_specs=pl.BlockSpec((ROW_BLK, 2), lambda i: (i, 0)),
        out_shape=jax.ShapeDtypeStruct((B // 2, 2), jnp.float32),
    )(h, W1d, b1d, W2d, b2d)


@jax.jit
def kernel(group_inputs, user_inputs, item_inputs, user_table, item_table,
           W1, b1, W2, b2):
    del group_inputs
    uidx = user_inputs.astype(jnp.int32)
    iidx = item_inputs.astype(jnp.int32)
    h = _sc_gather_mul(uidx, iidx, user_table.T, item_table.T)
    zero = jnp.zeros((D, 16), jnp.float32)
    W1d = jnp.concatenate(
        [jnp.concatenate([W1, zero], axis=1),
         jnp.concatenate([zero, W1], axis=1)], axis=0)
    b1d = jnp.concatenate([b1, b1]).reshape(1, 32)
    zero2 = jnp.zeros((16, 1), jnp.float32)
    W2d = jnp.concatenate(
        [jnp.concatenate([W2, zero2], axis=1),
         jnp.concatenate([zero2, W2], axis=1)], axis=0)
    b2d = jnp.concatenate([b2, b2]).reshape(1, 2)
    o = _tc_mlp(h, W1d, b1d, W2d, b2d)
    return o.reshape(B, 1)
